# baseline (device time: 112730 ns/iter reference)
import jax
import jax.numpy as jnp
from jax import lax
from jax.experimental import pallas as pl
from jax.experimental.pallas import tpu as pltpu

NCHUNK = 8
QCLIP = 4.5
QSCALE = 127.0 / QCLIP


def kernel(x, pi):
    _, m, n = x.shape
    rows = m // NCHUNK

    def body(x_ref, pi_ref, out_ref, fbuf, qbuf, rbuf, dbuf,
             fetch_sems, send_sems, recv_sems, store_sems):
        my_x = lax.axis_index("x")
        my_y = lax.axis_index("y")
        my_z = lax.axis_index("z")
        dst_z = pi_ref[my_z]
        src_z = jnp.where(
            pi_ref[0] == my_z, 0,
            jnp.where(pi_ref[1] == my_z, 1,
                      jnp.where(pi_ref[2] == my_z, 2, 3)))

        barrier_sem = pltpu.get_barrier_semaphore()
        pl.semaphore_signal(
            barrier_sem, inc=1, device_id=(my_x, my_y, dst_z),
            device_id_type=pl.DeviceIdType.MESH)
        pl.semaphore_signal(
            barrier_sem, inc=1, device_id=(my_x, my_y, src_z),
            device_id_type=pl.DeviceIdType.MESH)

        def fetch(i, slot):
            return pltpu.make_async_copy(
                x_ref.at[0, pl.ds(i * rows, rows), :],
                fbuf.at[slot],
                fetch_sems.at[slot],
            )

        def store(i, slot):
            return pltpu.make_async_copy(
                dbuf.at[slot],
                out_ref.at[0, pl.ds(i * rows, rows), :],
                store_sems.at[slot],
            )

        rdmas = [
            pltpu.make_async_remote_copy(
                src_ref=qbuf.at[i],
                dst_ref=rbuf.at[i],
                send_sem=send_sems.at[i],
                recv_sem=recv_sems.at[i],
                device_id=(my_x, my_y, dst_z),
                device_id_type=pl.DeviceIdType.MESH,
            )
            for i in range(NCHUNK)
        ]

        fetch(0, 0).start()
        fetch(1, 1).start()
        for i in range(NCHUNK):
            slot = i % 2
            fetch(i, slot).wait()
            qbuf[i] = jnp.clip(
                jnp.rint(fbuf[slot] * QSCALE), -127.0, 127.0
            ).astype(jnp.int8)
            if i + 2 < NCHUNK:
                fetch(i + 2, slot).start()
            if i == 0:
                pl.semaphore_wait(barrier_sem, 2)
            rdmas[i].start()

        for i in range(NCHUNK):
            slot = i % 2
            rdmas[i].wait_recv()
            if i >= 2:
                store(i - 2, slot).wait()
            dbuf[slot] = (
                rbuf[i].astype(jnp.float32) * (QCLIP / 127.0)
            ).astype(jnp.bfloat16)
            store(i, slot).start()

        for i in range(NCHUNK):
            rdmas[i].wait_send()
        store(NCHUNK - 2, (NCHUNK - 2) % 2).wait()
        store(NCHUNK - 1, (NCHUNK - 1) % 2).wait()

    return pl.pallas_call(
        body,
        out_shape=jax.ShapeDtypeStruct(x.shape, jnp.bfloat16),
        in_specs=[
            pl.BlockSpec(memory_space=pl.ANY),
            pl.BlockSpec(memory_space=pltpu.SMEM),
        ],
        out_specs=pl.BlockSpec(memory_space=pl.ANY),
        scratch_shapes=[
            pltpu.VMEM((2, rows, n), x.dtype),
            pltpu.VMEM((NCHUNK, rows, n), jnp.int8),
            pltpu.VMEM((NCHUNK, rows, n), jnp.int8),
            pltpu.VMEM((2, rows, n), jnp.bfloat16),
            pltpu.SemaphoreType.DMA((2,)),
            pltpu.SemaphoreType.DMA((NCHUNK,)),
            pltpu.SemaphoreType.DMA((NCHUNK,)),
            pltpu.SemaphoreType.DMA((2,)),
        ],
        compiler_params=pltpu.CompilerParams(
            vmem_limit_bytes=100 * 1024 * 1024,
            collective_id=0,
        ),
    )(x, pi)


# device time: 110793 ns/iter; 1.0175x vs baseline; 1.0175x over previous
import jax
import jax.numpy as jnp
from jax import lax
from jax.experimental import pallas as pl
from jax.experimental.pallas import tpu as pltpu

ROWS = (256, 512, 1024, 1024, 512, 384, 256, 128)
NCHUNK = len(ROWS)
MAXROWS = max(ROWS)
OFFS = tuple(sum(ROWS[:i]) for i in range(NCHUNK))

QCLIP = 4.5
QSCALE = 127.0 / QCLIP


def kernel(x, pi):
    _, m, n = x.shape
    assert m == sum(ROWS)

    def body(x_ref, pi_ref, out_ref, fbuf, qbuf, rbuf, dbuf,
             fetch_sems, send_sems, recv_sems, store_sems):
        my_x = lax.axis_index("x")
        my_y = lax.axis_index("y")
        my_z = lax.axis_index("z")
        dst_z = pi_ref[my_z]
        src_z = jnp.where(
            pi_ref[0] == my_z, 0,
            jnp.where(pi_ref[1] == my_z, 1,
                      jnp.where(pi_ref[2] == my_z, 2, 3)))

        barrier_sem = pltpu.get_barrier_semaphore()
        pl.semaphore_signal(
            barrier_sem, inc=1, device_id=(my_x, my_y, dst_z),
            device_id_type=pl.DeviceIdType.MESH)
        pl.semaphore_signal(
            barrier_sem, inc=1, device_id=(my_x, my_y, src_z),
            device_id_type=pl.DeviceIdType.MESH)

        def fetch(i, slot):
            return pltpu.make_async_copy(
                x_ref.at[0, pl.ds(OFFS[i], ROWS[i]), :],
                fbuf.at[slot, pl.ds(0, ROWS[i]), :],
                fetch_sems.at[slot],
            )

        def store(i, slot):
            return pltpu.make_async_copy(
                dbuf.at[slot, pl.ds(0, ROWS[i]), :],
                out_ref.at[0, pl.ds(OFFS[i], ROWS[i]), :],
                store_sems.at[slot],
            )

        rdmas = [
            pltpu.make_async_remote_copy(
                src_ref=qbuf.at[pl.ds(OFFS[i], ROWS[i]), :],
                dst_ref=rbuf.at[pl.ds(OFFS[i], ROWS[i]), :],
                send_sem=send_sems.at[i],
                recv_sem=recv_sems.at[i],
                device_id=(my_x, my_y, dst_z),
                device_id_type=pl.DeviceIdType.MESH,
            )
            for i in range(NCHUNK)
        ]

        fetch(0, 0).start()
        fetch(1, 1).start()
        for i in range(NCHUNK):
            slot = i % 2
            fetch(i, slot).wait()
            qbuf[pl.ds(OFFS[i], ROWS[i]), :] = jnp.clip(
                jnp.rint(fbuf[slot, pl.ds(0, ROWS[i]), :] * QSCALE),
                -127.0, 127.0,
            ).astype(jnp.int8)
            if i + 2 < NCHUNK:
                fetch(i + 2, slot).start()
            if i == 0:
                pl.semaphore_wait(barrier_sem, 2)
            rdmas[i].start()

        for i in range(NCHUNK):
            slot = i % 2
            rdmas[i].wait_recv()
            if i >= 2:
                store(i - 2, slot).wait()
            dbuf[slot, pl.ds(0, ROWS[i]), :] = (
                rbuf[pl.ds(OFFS[i], ROWS[i]), :].astype(jnp.float32)
                * (QCLIP / 127.0)
            ).astype(jnp.bfloat16)
            store(i, slot).start()

        for i in range(NCHUNK):
            rdmas[i].wait_send()
        store(NCHUNK - 2, (NCHUNK - 2) % 2).wait()
        store(NCHUNK - 1, (NCHUNK - 1) % 2).wait()

    return pl.pallas_call(
        body,
        out_shape=jax.ShapeDtypeStruct(x.shape, jnp.bfloat16),
        in_specs=[
            pl.BlockSpec(memory_space=pl.ANY),
            pl.BlockSpec(memory_space=pltpu.SMEM),
        ],
        out_specs=pl.BlockSpec(memory_space=pl.ANY),
        scratch_shapes=[
            pltpu.VMEM((2, MAXROWS, n), x.dtype),
            pltpu.VMEM((m, n), jnp.int8),
            pltpu.VMEM((m, n), jnp.int8),
            pltpu.VMEM((2, MAXROWS, n), jnp.bfloat16),
            pltpu.SemaphoreType.DMA((2,)),
            pltpu.SemaphoreType.DMA((NCHUNK,)),
            pltpu.SemaphoreType.DMA((NCHUNK,)),
            pltpu.SemaphoreType.DMA((2,)),
        ],
        compiler_params=pltpu.CompilerParams(
            vmem_limit_bytes=100 * 1024 * 1024,
            collective_id=0,
        ),
    )(x, pi)
